# baseline (speedup 1.0000x reference)
import functools
import jax
import jax.numpy as jnp
from jax import lax
from jax.experimental import pallas as pl
from jax.experimental.pallas import tpu as pltpu
from jax.experimental.pallas import tpu_sc as plsc

N = 10000
D = 512
K = 2048

_RB = 128
_NRB = (N + _RB - 1) // _RB
_WINDOWS = [(0, 16), (16, 16), (32, 16), (48, 16), (64, 15)]


def _degrees_body(a_ref, o_ref):
    x = a_ref[...]
    tot = None
    for (c0, nch) in _WINDOWS:
        p = None
        for t in range(nch):
            lo = (c0 + t) * 128
            hi = lo + 128
            if hi <= N:
                c = x[:, lo:hi]
            else:
                c = jnp.concatenate(
                    [x[:, lo:N], jnp.zeros((_RB, hi - N), jnp.float32)], axis=1)
            p = c if p is None else p + c
        t_p = p.T
        acc = t_p[0:8, :]
        for k in range(1, 16):
            acc = acc + t_p[8 * k:8 * k + 8, :]
        acc = acc[0:4, :] + acc[4:8, :]
        acc = acc[0:2, :] + acc[2:4, :]
        acc = acc[0:1, :] + acc[1:2, :]
        tot = acc if tot is None else tot + acc
    o_ref[...] = tot[None]


def _degrees(adjacency):
    out = pl.pallas_call(
        _degrees_body,
        grid=(_NRB,),
        in_specs=[pl.BlockSpec((_RB, N), lambda i: (i, 0))],
        out_specs=pl.BlockSpec((1, 1, _RB), lambda i: (i, 0, 0)),
        out_shape=jax.ShapeDtypeStruct((_NRB, 1, _RB), jnp.float32),
    )(adjacency)
    return out.reshape(_NRB * _RB)[:N]


_NC, _NS, _L = 2, 16, 16
_NW = _NC * _NS
_RPW = K // _NW


_RG = 8
_XG = 32


def _gather_body(x_hbm, adj_hbm, idx_hbm, xout_hbm, adjout_hbm,
                 idxall_v, myidx_v, rows_v, out_v, xrows_v, sem, sem2):
    wid = lax.axis_index("s") * _NC + lax.axis_index("c")
    base = wid * _RPW
    pltpu.sync_copy(idx_hbm.at[pl.ds(base, _RPW)], myidx_v)
    pltpu.sync_copy(idx_hbm, idxall_v)

    def x_step(j, carry):
        pltpu.async_copy(x_hbm.at[myidx_v.at[pl.ds(j * _XG, _XG)]],
                         xrows_v, sem2).wait()
        pltpu.sync_copy(xrows_v, xout_hbm.at[pl.ds(base + j * _XG, _XG)])
        return carry

    lax.fori_loop(0, _RPW // _XG, x_step, 0, unroll=True)

    def row_step(r, carry):
        iv = plsc.load_gather(myidx_v, [lax.broadcast(r, (_L,))])
        ridx = jnp.max(iv)
        pltpu.sync_copy(adj_hbm.at[pl.ds(ridx, 1)], rows_v)

        def col_step(c, carry2):
            cidx = idxall_v[pl.ds(c * _L, _L)]
            vals = plsc.load_gather(
                rows_v, [jnp.zeros((_L,), jnp.int32), cidx])
            out_v[pl.ds(c * _L, _L)] = vals
            return carry2

        lax.fori_loop(0, K // _L, col_step, 0, unroll=8)
        pltpu.sync_copy(out_v, adjout_hbm.at[pl.ds((base + r) * K, K)])
        return carry

    lax.fori_loop(0, _RPW, row_step, 0, unroll=False)


def _gather(x, adjacency, idx):
    mesh = plsc.VectorSubcoreMesh(
        core_axis_name="c", subcore_axis_name="s",
        num_cores=_NC, num_subcores=_NS)
    f = pl.kernel(
        _gather_body,
        out_type=(
            jax.ShapeDtypeStruct((K, D), jnp.float32),
            jax.ShapeDtypeStruct((K * K,), jnp.float32),
        ),
        mesh=mesh,
        compiler_params=pltpu.CompilerParams(needs_layout_passes=False),
        scratch_types=[
            pltpu.VMEM((K,), jnp.int32),
            pltpu.VMEM((_RPW,), jnp.int32),
            pltpu.VMEM((1, N), jnp.float32),
            pltpu.VMEM((K,), jnp.float32),
            pltpu.VMEM((_XG, D), jnp.float32),
            pltpu.SemaphoreType.DMA,
            pltpu.SemaphoreType.DMA,
        ],
    )
    x_pooled, adj_flat = f(x, adjacency, idx)
    return x_pooled, adj_flat.reshape(K, K)


def kernel(x, adjacency):
    degrees = _degrees(adjacency)
    _, idx = jax.lax.top_k(degrees, K)
    x_pooled, adj_pooled = _gather(x, adjacency, idx)
    return (x_pooled, adj_pooled)


# speedup vs baseline: 1.0000x; 1.0000x over previous
import jax
import jax.numpy as jnp
from jax import lax
from jax.experimental import pallas as pl
from jax.experimental.pallas import tpu as pltpu
from jax.experimental.pallas import tpu_sc as plsc

N = 10000
D = 512
K = 2048

_RB = 128
_NRB = (N + _RB - 1) // _RB
_WINDOWS = [(0, 16), (16, 16), (32, 16), (48, 16), (64, 15)]


def _degrees_body(a_ref, o_ref):
    x = a_ref[...]
    tot = None
    for (c0, nch) in _WINDOWS:
        p = None
        for t in range(nch):
            lo = (c0 + t) * 128
            hi = lo + 128
            if hi <= N:
                c = x[:, lo:hi]
            else:
                c = jnp.concatenate(
                    [x[:, lo:N], jnp.zeros((_RB, hi - N), jnp.float32)], axis=1)
            p = c if p is None else p + c
        t_p = p.T
        acc = t_p[0:8, :]
        for k in range(1, 16):
            acc = acc + t_p[8 * k:8 * k + 8, :]
        acc = acc[0:4, :] + acc[4:8, :]
        acc = acc[0:2, :] + acc[2:4, :]
        acc = acc[0:1, :] + acc[1:2, :]
        tot = acc if tot is None else tot + acc
    o_ref[...] = tot[None]


def _degrees(adjacency):
    out = pl.pallas_call(
        _degrees_body,
        grid=(_NRB,),
        in_specs=[pl.BlockSpec((_RB, N), lambda i: (i, 0))],
        out_specs=pl.BlockSpec((1, 1, _RB), lambda i: (i, 0, 0)),
        out_shape=jax.ShapeDtypeStruct((_NRB, 1, _RB), jnp.float32),
    )(adjacency)
    return out.reshape(_NRB * _RB)[:N]


_NC, _NS, _L = 2, 16, 16
_NW = _NC * _NS
_RPW = K // _NW
_XG = 32


def _gather_body(x_hbm, adj_hbm, idx_hbm, xout_hbm, adjout_hbm,
                 idxall_v, myidx_v, rows_v, out_v, xrows_v, sem, sem2):
    wid = lax.axis_index("s") * _NC + lax.axis_index("c")
    base = wid * _RPW
    pltpu.async_copy(idx_hbm.at[pl.ds(base, _RPW)], myidx_v, sem2).wait()
    pltpu.async_copy(idx_hbm, idxall_v, sem2).wait()

    def x_step(j, carry):
        pltpu.async_copy(x_hbm.at[myidx_v.at[pl.ds(j * _XG, _XG)]],
                         xrows_v, sem2).wait()
        pltpu.async_copy(xrows_v, xout_hbm.at[pl.ds(base + j * _XG, _XG)],
                         sem2).wait()
        return carry

    lax.fori_loop(0, _RPW // _XG, x_step, 0, unroll=True)

    def _ridx(r):
        iv = plsc.load_gather(myidx_v, [lax.broadcast(r, (_L,))])
        return jnp.max(iv)

    def row_step(r, carry):
        pltpu.async_copy(adj_hbm.at[pl.ds(_ridx(r), 1)], rows_v, sem).wait()

        def col_step(c, carry2):
            cidx = idxall_v[pl.ds(c * _L, _L)]
            vals = plsc.load_gather(
                rows_v, [jnp.zeros((_L,), jnp.int32), cidx])
            out_v[pl.ds(c * _L, _L)] = vals
            return carry2

        lax.fori_loop(0, K // _L, col_step, 0, unroll=8)
        pltpu.async_copy(out_v, adjout_hbm.at[pl.ds((base + r) * K, K)],
                         sem).wait()
        return carry

    lax.fori_loop(0, _RPW, row_step, 0, unroll=False)


def _gather(x, adjacency, idx):
    mesh = plsc.VectorSubcoreMesh(
        core_axis_name="c", subcore_axis_name="s",
        num_cores=_NC, num_subcores=_NS)
    f = pl.kernel(
        _gather_body,
        out_type=(
            jax.ShapeDtypeStruct((K, D), jnp.float32),
            jax.ShapeDtypeStruct((K * K,), jnp.float32),
        ),
        mesh=mesh,
        compiler_params=pltpu.CompilerParams(needs_layout_passes=False),
        scratch_types=[
            pltpu.VMEM((K,), jnp.int32),
            pltpu.VMEM((_RPW,), jnp.int32),
            pltpu.VMEM((1, N), jnp.float32),
            pltpu.VMEM((K,), jnp.float32),
            pltpu.VMEM((_XG, D), jnp.float32),
            pltpu.SemaphoreType.DMA,
            pltpu.SemaphoreType.DMA,
        ],
    )
    x_pooled, adj_flat = f(x, adjacency, idx)
    return x_pooled, adj_flat.reshape(K, K)


def kernel(x, adjacency):
    degrees = _degrees(adjacency)
    _, idx = jax.lax.top_k(degrees, K)
    x_pooled, adj_pooled = _gather(x, adjacency, idx)
    return (x_pooled, adj_pooled)


# speedup vs baseline: 1.0247x; 1.0247x over previous
import jax
import jax.numpy as jnp
from jax import lax
from jax.experimental import pallas as pl
from jax.experimental.pallas import tpu as pltpu
from jax.experimental.pallas import tpu_sc as plsc

N = 10000
D = 512
K = 2048

_RB = 128
_NRB = (N + _RB - 1) // _RB
_WINDOWS = [(0, 16), (16, 16), (32, 16), (48, 16), (64, 15)]


def _degrees_body(a_ref, o_ref):
    x = a_ref[...]
    tot = None
    for (c0, nch) in _WINDOWS:
        p = None
        for t in range(nch):
            lo = (c0 + t) * 128
            hi = lo + 128
            if hi <= N:
                c = x[:, lo:hi]
            else:
                c = jnp.concatenate(
                    [x[:, lo:N], jnp.zeros((_RB, hi - N), jnp.float32)], axis=1)
            p = c if p is None else p + c
        t_p = p.T
        acc = t_p[0:8, :]
        for k in range(1, 16):
            acc = acc + t_p[8 * k:8 * k + 8, :]
        acc = acc[0:4, :] + acc[4:8, :]
        acc = acc[0:2, :] + acc[2:4, :]
        acc = acc[0:1, :] + acc[1:2, :]
        tot = acc if tot is None else tot + acc
    o_ref[...] = tot[None]


def _degrees(adjacency):
    out = pl.pallas_call(
        _degrees_body,
        grid=(_NRB,),
        in_specs=[pl.BlockSpec((_RB, N), lambda i: (i, 0))],
        out_specs=pl.BlockSpec((1, 1, _RB), lambda i: (i, 0, 0)),
        out_shape=jax.ShapeDtypeStruct((_NRB, 1, _RB), jnp.float32),
    )(adjacency)
    return out.reshape(_NRB * _RB)[:N]


_NC, _NS, _L = 2, 16, 16
_NW = _NC * _NS
_RPW = K // _NW
_XG = 32


def _gather_body(x_hbm, adj_hbm, idx_hbm, xout_hbm, adjout_hbm, dummy_hbm,
                 idxall_v, myidx_v, rows_v, out_v, xrows_v, sem, sem_st, sem2):
    wid = lax.axis_index("s") * _NC + lax.axis_index("c")
    base = wid * _RPW
    pltpu.async_copy(idx_hbm.at[pl.ds(base, _RPW)], myidx_v, sem2).wait()
    pltpu.async_copy(idx_hbm, idxall_v, sem2).wait()

    def x_step(j, carry):
        pltpu.async_copy(x_hbm.at[myidx_v.at[pl.ds(j * _XG, _XG)]],
                         xrows_v, sem2).wait()
        pltpu.async_copy(xrows_v, xout_hbm.at[pl.ds(base + j * _XG, _XG)],
                         sem2).wait()
        return carry

    lax.fori_loop(0, _RPW // _XG, x_step, 0, unroll=True)

    def _ridx(r):
        iv = plsc.load_gather(myidx_v, [lax.broadcast(r, (_L,))])
        return jnp.max(iv)

    pltpu.async_copy(out_v, dummy_hbm.at[pl.ds(wid * K, K)], sem_st)

    def row_step(r, carry):
        pltpu.async_copy(adj_hbm.at[pl.ds(_ridx(r), 1)], rows_v, sem).wait()
        pltpu.make_async_copy(out_v, dummy_hbm.at[pl.ds(wid * K, K)],
                              sem_st).wait()

        def col_step(c, carry2):
            cidx = idxall_v[pl.ds(c * _L, _L)]
            vals = plsc.load_gather(
                rows_v, [jnp.zeros((_L,), jnp.int32), cidx])
            out_v[pl.ds(c * _L, _L)] = vals
            return carry2

        lax.fori_loop(0, K // _L, col_step, 0, unroll=8)
        pltpu.async_copy(out_v, adjout_hbm.at[pl.ds((base + r) * K, K)],
                         sem_st)
        return carry

    lax.fori_loop(0, _RPW, row_step, 0, unroll=False)
    pltpu.make_async_copy(out_v, dummy_hbm.at[pl.ds(wid * K, K)],
                          sem_st).wait()


def _gather(x, adjacency, idx):
    mesh = plsc.VectorSubcoreMesh(
        core_axis_name="c", subcore_axis_name="s",
        num_cores=_NC, num_subcores=_NS)
    f = pl.kernel(
        _gather_body,
        out_type=(
            jax.ShapeDtypeStruct((K, D), jnp.float32),
            jax.ShapeDtypeStruct((K * K,), jnp.float32),
            jax.ShapeDtypeStruct((_NW * K,), jnp.float32),
        ),
        mesh=mesh,
        compiler_params=pltpu.CompilerParams(needs_layout_passes=False),
        scratch_types=[
            pltpu.VMEM((K,), jnp.int32),
            pltpu.VMEM((_RPW,), jnp.int32),
            pltpu.VMEM((1, N), jnp.float32),
            pltpu.VMEM((K,), jnp.float32),
            pltpu.VMEM((_XG, D), jnp.float32),
            pltpu.SemaphoreType.DMA,
            pltpu.SemaphoreType.DMA,
            pltpu.SemaphoreType.DMA,
        ],
    )
    x_pooled, adj_flat, _ = f(x, adjacency, idx)
    return x_pooled, adj_flat.reshape(K, K)


def kernel(x, adjacency):
    degrees = _degrees(adjacency)
    _, idx = jax.lax.top_k(degrees, K)
    x_pooled, adj_pooled = _gather(x, adjacency, idx)
    return (x_pooled, adj_pooled)
